# Initial kernel scaffold; baseline (speedup 1.0000x reference)
import jax
import jax.numpy as jnp
from jax.experimental import pallas as pl


def kernel(x, indices, ln_in_g, ln_in_b, W_ga, b_ga, W_la, b_la, W_gb, b_gb, W_lb, b_lb, ln_o_g, ln_o_b, W_go, b_go, W_lo, b_lo):
    raise NotImplementedError("write your pallas kernel here")


# speedup vs baseline: 1.5318x; 1.5318x over previous
import functools

import jax
import jax.numpy as jnp
from jax import lax
from jax.experimental import pallas as pl
from jax.experimental.pallas import tpu as pltpu
from jax.experimental.pallas import tpu_sc as plsc

N = 50000
D = 128
C = 128
KNN = 8

NC = 2
NS = 16
NW = NC * NS
ROWS_W = 1600
NPAD = NW * ROWS_W
RB = 32
NBLK = ROWS_W // RB
IDX_ROWS = RB * KNN // 128
LANES = 16

_EPS = 1e-5


def _stage1_body(x_ref, wcat_ref, bcat_ref, g_ref, b_ref, a_ref, bb_ref, go_ref):
    x = x_ref[...]
    m = jnp.mean(x, axis=-1, keepdims=True)
    v = jnp.mean((x - m) * (x - m), axis=-1, keepdims=True)
    xn = (x - m) * lax.rsqrt(v + _EPS) * g_ref[...] + b_ref[...]
    y = jnp.dot(xn, wcat_ref[...], preferred_element_type=jnp.float32) + bcat_ref[...]
    a_ref[...] = jax.nn.sigmoid(y[:, 0:C]) * y[:, C:2 * C]
    bb_ref[...] = jax.nn.sigmoid(y[:, 2 * C:3 * C]) * y[:, 3 * C:4 * C]
    go_ref[...] = jax.nn.sigmoid(y[:, 4 * C:5 * C])


def _stage3_body(a_ref, s_ref, go_ref, wlo_ref, blo_ref, g_ref, b_ref, out_ref):
    k = a_ref[...] * s_ref[...]
    m = jnp.mean(k, axis=-1, keepdims=True)
    v = jnp.mean((k - m) * (k - m), axis=-1, keepdims=True)
    kn = (k - m) * lax.rsqrt(v + _EPS) * g_ref[...] + b_ref[...]
    y = jnp.dot(kn, wlo_ref[...], preferred_element_type=jnp.float32) + blo_ref[...]
    out_ref[...] = go_ref[...] * y


_BLK1 = 1000


def _stage1(x, wcat, bcat, g, b):
    grid = (N // _BLK1,)
    return pl.pallas_call(
        _stage1_body,
        grid=grid,
        in_specs=[
            pl.BlockSpec((_BLK1, D), lambda i: (i, 0)),
            pl.BlockSpec((D, 5 * C), lambda i: (0, 0)),
            pl.BlockSpec((1, 5 * C), lambda i: (0, 0)),
            pl.BlockSpec((1, D), lambda i: (0, 0)),
            pl.BlockSpec((1, D), lambda i: (0, 0)),
        ],
        out_specs=[
            pl.BlockSpec((_BLK1, C), lambda i: (i, 0)),
            pl.BlockSpec((_BLK1, C), lambda i: (i, 0)),
            pl.BlockSpec((_BLK1, C), lambda i: (i, 0)),
        ],
        out_shape=[
            jax.ShapeDtypeStruct((N, C), jnp.float32),
            jax.ShapeDtypeStruct((N, C), jnp.float32),
            jax.ShapeDtypeStruct((N, C), jnp.float32),
        ],
    )(x, wcat, bcat, g, b)


def _stage3(a, s, go, wlo, blo, g, b):
    grid = (N // _BLK1,)
    return pl.pallas_call(
        _stage3_body,
        grid=grid,
        in_specs=[
            pl.BlockSpec((_BLK1, C), lambda i: (i, 0)),
            pl.BlockSpec((_BLK1, C), lambda i: (i, 0)),
            pl.BlockSpec((_BLK1, C), lambda i: (i, 0)),
            pl.BlockSpec((C, D), lambda i: (0, 0)),
            pl.BlockSpec((1, D), lambda i: (0, 0)),
            pl.BlockSpec((1, C), lambda i: (0, 0)),
            pl.BlockSpec((1, C), lambda i: (0, 0)),
        ],
        out_specs=pl.BlockSpec((_BLK1, D), lambda i: (i, 0)),
        out_shape=jax.ShapeDtypeStruct((N, D), jnp.float32),
    )(a, s, go, wlo, blo, g, b)


def _gather_sum_body(bb_hbm, idx_hbm, out_hbm, idx_v, gat_v, acc_v, sem):
    cid = lax.axis_index("c")
    sid = lax.axis_index("s")
    wid = sid * NC + cid

    def blk_body(bi, carry):
        base = wid * ROWS_W + bi * RB
        idx_row0 = wid * (ROWS_W * KNN // 128) + bi * IDX_ROWS
        pltpu.sync_copy(idx_hbm.at[pl.ds(idx_row0, IDX_ROWS)], idx_v)
        cps = [
            pltpu.async_copy(
                bb_hbm.at[idx_v.at[g]],
                gat_v.at[pl.ds(g * 128, 128)],
                sem,
            )
            for g in range(IDX_ROWS)
        ]
        for cp in cps:
            cp.wait()

        def row_body(r, rcarry):
            for cseg in range(C // LANES):
                acc = gat_v[r * KNN, pl.ds(cseg * LANES, LANES)]
                for j in range(1, KNN):
                    acc = acc + gat_v[r * KNN + j, pl.ds(cseg * LANES, LANES)]
                acc_v[r, pl.ds(cseg * LANES, LANES)] = acc
            return rcarry

        lax.fori_loop(0, RB, row_body, 0)
        pltpu.sync_copy(acc_v, out_hbm.at[pl.ds(base, RB)])
        return carry

    lax.fori_loop(0, NBLK, blk_body, 0)


def _gather_sum(bb, idx2d):
    mesh = plsc.VectorSubcoreMesh(core_axis_name="c", subcore_axis_name="s")
    fn = functools.partial(
        pl.kernel,
        mesh=mesh,
        out_type=jax.ShapeDtypeStruct((NPAD, C), jnp.float32),
        scratch_types=[
            pltpu.VMEM((IDX_ROWS, 128), jnp.int32),
            pltpu.VMEM((RB * KNN, C), jnp.float32),
            pltpu.VMEM((RB, C), jnp.float32),
            pltpu.SemaphoreType.DMA,
        ],
    )(_gather_sum_body)
    return fn(bb, idx2d)


def kernel(x, indices, ln_in_g, ln_in_b, W_ga, b_ga, W_la, b_la,
           W_gb, b_gb, W_lb, b_lb, ln_o_g, ln_o_b, W_go, b_go, W_lo, b_lo):
    wcat = jnp.concatenate(
        [W_ga.T, W_la.T, W_gb.T, W_lb.T, W_go.T], axis=1)
    bcat = jnp.concatenate([b_ga, b_la, b_gb, b_lb, b_go])[None, :]
    a, bb, go = _stage1(x, wcat, bcat, ln_in_g[None, :], ln_in_b[None, :])

    idx = indices[0].reshape(-1)
    idx = jnp.pad(idx, (0, (NPAD - N) * KNN)).reshape(-1, 128)
    s = _gather_sum(bb, idx)[:N]

    return _stage3(a, s, go, W_lo.T, b_lo[None, :],
                   ln_o_g[None, :], ln_o_b[None, :])


# speedup vs baseline: 1.6520x; 1.0785x over previous
import functools

import jax
import jax.numpy as jnp
from jax import lax
from jax.experimental import pallas as pl
from jax.experimental.pallas import tpu as pltpu
from jax.experimental.pallas import tpu_sc as plsc

N = 50000
D = 128
C = 128
KNN = 8

NC = 2
NS = 16
NW = NC * NS
ROWS_W = 1600
NPAD = NW * ROWS_W
RB = 32
NBLK = ROWS_W // RB
IDX_ROWS = RB * KNN // 128
LANES = 16

_EPS = 1e-5


def _stage1_body(x_ref, wcat_ref, bcat_ref, g_ref, b_ref, a_ref, bb_ref, go_ref):
    x = x_ref[...]
    m = jnp.mean(x, axis=-1, keepdims=True)
    v = jnp.mean((x - m) * (x - m), axis=-1, keepdims=True)
    xn = (x - m) * lax.rsqrt(v + _EPS) * g_ref[...] + b_ref[...]
    y = jnp.dot(xn, wcat_ref[...], preferred_element_type=jnp.float32) + bcat_ref[...]
    a_ref[...] = jax.nn.sigmoid(y[:, 0:C]) * y[:, C:2 * C]
    bb_ref[...] = jax.nn.sigmoid(y[:, 2 * C:3 * C]) * y[:, 3 * C:4 * C]
    go_ref[...] = jax.nn.sigmoid(y[:, 4 * C:5 * C])


def _stage3_body(a_ref, s_ref, go_ref, wlo_ref, blo_ref, g_ref, b_ref, out_ref):
    k = a_ref[...] * s_ref[...]
    m = jnp.mean(k, axis=-1, keepdims=True)
    v = jnp.mean((k - m) * (k - m), axis=-1, keepdims=True)
    kn = (k - m) * lax.rsqrt(v + _EPS) * g_ref[...] + b_ref[...]
    y = jnp.dot(kn, wlo_ref[...], preferred_element_type=jnp.float32) + blo_ref[...]
    out_ref[...] = go_ref[...] * y


_BLK1 = 1000


def _stage1(x, wcat, bcat, g, b):
    grid = (N // _BLK1,)
    return pl.pallas_call(
        _stage1_body,
        grid=grid,
        in_specs=[
            pl.BlockSpec((_BLK1, D), lambda i: (i, 0)),
            pl.BlockSpec((D, 5 * C), lambda i: (0, 0)),
            pl.BlockSpec((1, 5 * C), lambda i: (0, 0)),
            pl.BlockSpec((1, D), lambda i: (0, 0)),
            pl.BlockSpec((1, D), lambda i: (0, 0)),
        ],
        out_specs=[
            pl.BlockSpec((_BLK1, C), lambda i: (i, 0)),
            pl.BlockSpec((_BLK1, C), lambda i: (i, 0)),
            pl.BlockSpec((_BLK1, C), lambda i: (i, 0)),
        ],
        out_shape=[
            jax.ShapeDtypeStruct((N, C), jnp.float32),
            jax.ShapeDtypeStruct((N, C), jnp.float32),
            jax.ShapeDtypeStruct((N, C), jnp.float32),
        ],
    )(x, wcat, bcat, g, b)


def _stage3(a, s, go, wlo, blo, g, b):
    grid = (N // _BLK1,)
    return pl.pallas_call(
        _stage3_body,
        grid=grid,
        in_specs=[
            pl.BlockSpec((_BLK1, C), lambda i: (i, 0)),
            pl.BlockSpec((_BLK1, C), lambda i: (i, 0)),
            pl.BlockSpec((_BLK1, C), lambda i: (i, 0)),
            pl.BlockSpec((C, D), lambda i: (0, 0)),
            pl.BlockSpec((1, D), lambda i: (0, 0)),
            pl.BlockSpec((1, C), lambda i: (0, 0)),
            pl.BlockSpec((1, C), lambda i: (0, 0)),
        ],
        out_specs=pl.BlockSpec((_BLK1, D), lambda i: (i, 0)),
        out_shape=jax.ShapeDtypeStruct((N, D), jnp.float32),
    )(a, s, go, wlo, blo, g, b)


def _gather_sum_body(bb_hbm, idx_hbm, out_hbm,
                     idx0, idx1, gat0, gat1, acc0, acc1,
                     sg0, sg1, si0, si1, so0, so1):
    cid = lax.axis_index("c")
    sid = lax.axis_index("s")
    wid = sid * NC + cid
    idx_v = (idx0, idx1)
    gat_v = (gat0, gat1)
    acc_v = (acc0, acc1)
    sem_g = (sg0, sg1)
    sem_i = (si0, si1)
    sem_o = (so0, so1)
    idx_base = wid * (ROWS_W * KNN // 128)

    def fire_idx(bi, k):
        pltpu.async_copy(
            idx_hbm.at[pl.ds(idx_base + bi * IDX_ROWS, IDX_ROWS)],
            idx_v[k], sem_i[k])

    def wait_idx(k):
        pltpu.make_async_copy(
            idx_hbm.at[pl.ds(0, IDX_ROWS)], idx_v[k], sem_i[k]).wait()

    def fire_gathers(k):
        for g in range(IDX_ROWS):
            pltpu.async_copy(
                bb_hbm.at[idx_v[k].at[g]],
                gat_v[k].at[pl.ds(g * 128, 128)], sem_g[k])

    def wait_gathers(k):
        for g in range(IDX_ROWS):
            pltpu.make_async_copy(
                bb_hbm.at[pl.ds(0, 128)],
                gat_v[k].at[pl.ds(g * 128, 128)], sem_g[k]).wait()

    def wait_out(k):
        pltpu.make_async_copy(
            acc_v[k], out_hbm.at[pl.ds(0, RB)], sem_o[k]).wait()

    def compute(k):
        def row_body(r2, rcarry):
            for u in range(2):
                r = r2 * 2 + u
                for cseg in range(C // LANES):
                    acc = gat_v[k][r * KNN, pl.ds(cseg * LANES, LANES)]
                    for j in range(1, KNN):
                        acc = acc + gat_v[k][r * KNN + j,
                                            pl.ds(cseg * LANES, LANES)]
                    acc_v[k][r, pl.ds(cseg * LANES, LANES)] = acc
            return rcarry
        lax.fori_loop(0, RB // 2, row_body, 0)

    for k in (0, 1):
        pltpu.async_copy(bb_hbm.at[pl.ds(0, RB)], acc_v[k], sem_o[k])
        fire_idx(k, k)
    for k in (0, 1):
        wait_idx(k)
        fire_gathers(k)

    def it_body(it, carry):
        for b in (0, 1):
            bi = it * 2 + b
            nxt = jnp.minimum(bi + 2, NBLK - 1)
            wait_gathers(b)
            fire_idx(nxt, b)
            wait_out(b)
            compute(b)
            pltpu.async_copy(
                acc_v[b], out_hbm.at[pl.ds(wid * ROWS_W + bi * RB, RB)],
                sem_o[b])
            wait_idx(b)
            fire_gathers(b)
        return carry

    lax.fori_loop(0, NBLK // 2, it_body, 0)

    for k in (0, 1):
        wait_gathers(k)
        wait_out(k)


def _gather_sum(bb, idx2d):
    mesh = plsc.VectorSubcoreMesh(core_axis_name="c", subcore_axis_name="s")
    fn = functools.partial(
        pl.kernel,
        mesh=mesh,
        out_type=jax.ShapeDtypeStruct((NPAD, C), jnp.float32),
        scratch_types=[
            pltpu.VMEM((IDX_ROWS, 128), jnp.int32),
            pltpu.VMEM((IDX_ROWS, 128), jnp.int32),
            pltpu.VMEM((RB * KNN, C), jnp.float32),
            pltpu.VMEM((RB * KNN, C), jnp.float32),
            pltpu.VMEM((RB, C), jnp.float32),
            pltpu.VMEM((RB, C), jnp.float32),
            pltpu.SemaphoreType.DMA,
            pltpu.SemaphoreType.DMA,
            pltpu.SemaphoreType.DMA,
            pltpu.SemaphoreType.DMA,
            pltpu.SemaphoreType.DMA,
            pltpu.SemaphoreType.DMA,
        ],
    )(_gather_sum_body)
    return fn(bb, idx2d)


def kernel(x, indices, ln_in_g, ln_in_b, W_ga, b_ga, W_la, b_la,
           W_gb, b_gb, W_lb, b_lb, ln_o_g, ln_o_b, W_go, b_go, W_lo, b_lo):
    wcat = jnp.concatenate(
        [W_ga.T, W_la.T, W_gb.T, W_lb.T, W_go.T], axis=1)
    bcat = jnp.concatenate([b_ga, b_la, b_gb, b_lb, b_go])[None, :]
    a, bb, go = _stage1(x, wcat, bcat, ln_in_g[None, :], ln_in_b[None, :])

    idx = indices[0].reshape(-1)
    idx = jnp.pad(idx, (0, (NPAD - N) * KNN)).reshape(-1, 128)
    s = _gather_sum(bb, idx)[:N]

    return _stage3(a, s, go, W_lo.T, b_lo[None, :],
                   ln_o_g[None, :], ln_o_b[None, :])


# speedup vs baseline: 3.0012x; 1.8167x over previous
import functools

import jax
import jax.numpy as jnp
from jax import lax
from jax.experimental import pallas as pl
from jax.experimental.pallas import tpu as pltpu
from jax.experimental.pallas import tpu_sc as plsc

N = 50000
D = 128
C = 128
KNN = 8

NC = 2
NS = 16
RB = 32
ROWS_W0 = 2560
ROWS_W1 = 640
NOUT = NS * (ROWS_W0 + ROWS_W1)
IDX_ROWS = RB * KNN // 128
IDXW_ROWS = ROWS_W0 * KNN // 128
LANES = 16

_EPS = 1e-5


def _stage1_body(x_ref, wcat_ref, bcat_ref, g_ref, b_ref, a_ref, bb_ref, go_ref):
    x = x_ref[...]
    m = jnp.mean(x, axis=-1, keepdims=True)
    v = jnp.mean((x - m) * (x - m), axis=-1, keepdims=True)
    xn = (x - m) * lax.rsqrt(v + _EPS) * g_ref[...] + b_ref[...]
    y = jnp.dot(xn, wcat_ref[...], preferred_element_type=jnp.float32) + bcat_ref[...]
    a_ref[...] = jax.nn.sigmoid(y[:, 0:C]) * y[:, C:2 * C]
    bb_ref[...] = jax.nn.sigmoid(y[:, 2 * C:3 * C]) * y[:, 3 * C:4 * C]
    go_ref[...] = jax.nn.sigmoid(y[:, 4 * C:5 * C])


def _stage3_body(a_ref, s_ref, go_ref, wlo_ref, blo_ref, g_ref, b_ref, out_ref):
    k = a_ref[...] * s_ref[...]
    m = jnp.mean(k, axis=-1, keepdims=True)
    v = jnp.mean((k - m) * (k - m), axis=-1, keepdims=True)
    kn = (k - m) * lax.rsqrt(v + _EPS) * g_ref[...] + b_ref[...]
    y = jnp.dot(kn, wlo_ref[...], preferred_element_type=jnp.float32) + blo_ref[...]
    out_ref[...] = go_ref[...] * y


_BLK1 = 1000


def _stage1(x, wcat, bcat, g, b):
    grid = (N // _BLK1,)
    return pl.pallas_call(
        _stage1_body,
        grid=grid,
        in_specs=[
            pl.BlockSpec((_BLK1, D), lambda i: (i, 0)),
            pl.BlockSpec((D, 5 * C), lambda i: (0, 0)),
            pl.BlockSpec((1, 5 * C), lambda i: (0, 0)),
            pl.BlockSpec((1, D), lambda i: (0, 0)),
            pl.BlockSpec((1, D), lambda i: (0, 0)),
        ],
        out_specs=[
            pl.BlockSpec((_BLK1, C), lambda i: (i, 0)),
            pl.BlockSpec((_BLK1, C), lambda i: (i, 0)),
            pl.BlockSpec((_BLK1, C), lambda i: (i, 0)),
        ],
        out_shape=[
            jax.ShapeDtypeStruct((N, C), jnp.float32),
            jax.ShapeDtypeStruct((N, C), jnp.float32),
            jax.ShapeDtypeStruct((N, C), jnp.float32),
        ],
    )(x, wcat, bcat, g, b)


def _stage3(a, s, go, wlo, blo, g, b):
    grid = (N // _BLK1,)
    return pl.pallas_call(
        _stage3_body,
        grid=grid,
        in_specs=[
            pl.BlockSpec((_BLK1, C), lambda i: (i, 0)),
            pl.BlockSpec((_BLK1, C), lambda i: (i, 0)),
            pl.BlockSpec((_BLK1, C), lambda i: (i, 0)),
            pl.BlockSpec((C, D), lambda i: (0, 0)),
            pl.BlockSpec((1, D), lambda i: (0, 0)),
            pl.BlockSpec((1, C), lambda i: (0, 0)),
            pl.BlockSpec((1, C), lambda i: (0, 0)),
        ],
        out_specs=pl.BlockSpec((_BLK1, D), lambda i: (i, 0)),
        out_shape=jax.ShapeDtypeStruct((N, D), jnp.float32),
    )(a, s, go, wlo, blo, g, b)


def _gather_sum_body(bb_hbm, idx_hbm, out_hbm,
                     idx_v, gat0, gat1, acc0, acc1,
                     sg0, sg1, so0, so1):
    cid = lax.axis_index("c")
    sid = lax.axis_index("s")
    gat_v = (gat0, gat1)
    acc_v = (acc0, acc1)
    sem_g = (sg0, sg1)
    sem_o = (so0, so1)
    rows_base = pl.multiple_of(
        jnp.where(cid == 0, sid * ROWS_W0, NS * ROWS_W0 + sid * ROWS_W1), RB)
    nblk = jnp.where(cid == 0, ROWS_W0 // RB, ROWS_W1 // RB)
    idx_base = pl.multiple_of(rows_base // (128 // KNN), 8)

    pltpu.sync_copy(idx_hbm.at[pl.ds(idx_base, IDXW_ROWS)], idx_v)

    def fire_gathers(bi, k):
        for g in range(IDX_ROWS):
            pltpu.async_copy(
                bb_hbm.at[idx_v.at[bi * IDX_ROWS + g]],
                gat_v[k].at[pl.ds(g * 128, 128)], sem_g[k])

    def wait_gathers(k):
        for g in range(IDX_ROWS):
            pltpu.make_async_copy(
                bb_hbm.at[pl.ds(0, 128)],
                gat_v[k].at[pl.ds(g * 128, 128)], sem_g[k]).wait()

    def wait_out(k):
        pltpu.make_async_copy(
            acc_v[k], out_hbm.at[pl.ds(0, RB)], sem_o[k]).wait()

    def compute(k):
        def row_body(r2, rcarry):
            for u in range(2):
                r = r2 * 2 + u
                for cseg in range(C // LANES):
                    acc = gat_v[k][r * KNN, pl.ds(cseg * LANES, LANES)]
                    for j in range(1, KNN):
                        acc = acc + gat_v[k][r * KNN + j,
                                            pl.ds(cseg * LANES, LANES)]
                    acc_v[k][r, pl.ds(cseg * LANES, LANES)] = acc
            return rcarry
        lax.fori_loop(0, RB // 2, row_body, 0)

    for k in (0, 1):
        pltpu.async_copy(bb_hbm.at[pl.ds(0, RB)], acc_v[k], sem_o[k])
        fire_gathers(k, k)

    def it_body(it, carry):
        for b in (0, 1):
            bi = it * 2 + b
            nxt = jnp.minimum(bi + 2, nblk - 1)
            wait_gathers(b)
            wait_out(b)
            compute(b)
            pltpu.async_copy(
                acc_v[b], out_hbm.at[pl.ds(rows_base + bi * RB, RB)],
                sem_o[b])
            fire_gathers(nxt, b)
        return carry

    lax.fori_loop(0, nblk // 2, it_body, 0)

    for k in (0, 1):
        wait_gathers(k)
        wait_out(k)


def _gather_sum(bb, idx2d):
    mesh = plsc.VectorSubcoreMesh(core_axis_name="c", subcore_axis_name="s")
    fn = functools.partial(
        pl.kernel,
        mesh=mesh,
        out_type=jax.ShapeDtypeStruct((NOUT, C), jnp.float32),
        scratch_types=[
            pltpu.VMEM((IDXW_ROWS, 128), jnp.int32),
            pltpu.VMEM((RB * KNN, C), jnp.float32),
            pltpu.VMEM((RB * KNN, C), jnp.float32),
            pltpu.VMEM((RB, C), jnp.float32),
            pltpu.VMEM((RB, C), jnp.float32),
            pltpu.SemaphoreType.DMA,
            pltpu.SemaphoreType.DMA,
            pltpu.SemaphoreType.DMA,
            pltpu.SemaphoreType.DMA,
        ],
    )(_gather_sum_body)
    return fn(bb, idx2d)


def kernel(x, indices, ln_in_g, ln_in_b, W_ga, b_ga, W_la, b_la,
           W_gb, b_gb, W_lb, b_lb, ln_o_g, ln_o_b, W_go, b_go, W_lo, b_lo):
    wcat = jnp.concatenate(
        [W_ga.T, W_la.T, W_gb.T, W_lb.T, W_go.T], axis=1)
    bcat = jnp.concatenate([b_ga, b_la, b_gb, b_lb, b_go])[None, :]
    a, bb, go = _stage1(x, wcat, bcat, ln_in_g[None, :], ln_in_b[None, :])

    idx2d = indices.reshape(2 * N * KNN // 128, 128)
    s = _gather_sum(bb, idx2d)

    return _stage3(a, s, go, W_lo.T, b_lo[None, :],
                   ln_o_g[None, :], ln_o_b[None, :])
